# baseline (speedup 1.0000x reference)
import functools

import jax
import jax.numpy as jnp
from jax import lax
from jax.experimental import pallas as pl
from jax.experimental.pallas import tpu as pltpu
from jax.experimental.pallas import tpu_sc as plsc

N_DIM = 16
L = 16
CHUNK = 128


def _rsqrt(x):
    xi = plsc.bitcast(x, jnp.int32)
    y = plsc.bitcast(jnp.int32(0x5F3759DF) - (xi >> 1), jnp.float32)
    for _ in range(3):
        y = y * (1.5 - 0.5 * x * y * y)
    return y


def _make_kernel(batch, n_rows):
    info = plsc.get_sparse_core_info()
    nc, ns = info.num_cores, info.num_subcores
    nw = nc * ns
    assert batch % (nw * L) == 0
    bpw = batch // nw
    n_chunks = bpw // CHUNK
    n_groups = bpw // L
    gpc = CHUNK // L

    mesh = plsc.VectorSubcoreMesh(core_axis_name="c", subcore_axis_name="s")

    @functools.partial(
        pl.kernel,
        mesh=mesh,
        compiler_params=pltpu.CompilerParams(
            needs_layout_passes=False, use_tc_tiling_on_sc=False),
        out_type=jax.ShapeDtypeStruct((batch,), jnp.float32),
        scratch_types=[
            pltpu.VMEM((n_chunks, CHUNK), jnp.int32),
            pltpu.VMEM((n_chunks, CHUNK), jnp.int32),
            pltpu.VMEM((N_DIM * bpw,), jnp.float32),
            pltpu.VMEM((N_DIM * bpw,), jnp.float32),
            pltpu.VMEM((bpw,), jnp.float32),
            pltpu.VMEM((bpw,), jnp.float32),
            pltpu.VMEM((L,), jnp.float32),
            pltpu.VMEM((bpw,), jnp.float32),
            pltpu.SemaphoreType.DMA,
        ],
    )
    def k(u_hbm, i_hbm, ub_hbm, uv_hbm, ib_hbm, iv_hbm, gb_hbm, out_hbm,
          u_v, i_v, pu_v, pi_v, ub_v, ib_v, gb_v, out_v, sem):
        wid = lax.axis_index("s") * nc + lax.axis_index("c")
        base = wid * bpw

        idx_cps = []
        for c in range(n_chunks):
            idx_cps.append(pltpu.async_copy(
                u_hbm.at[pl.ds(base + c * CHUNK, CHUNK)], u_v.at[c], sem))
            idx_cps.append(pltpu.async_copy(
                i_hbm.at[pl.ds(base + c * CHUNK, CHUNK)], i_v.at[c], sem))
        pltpu.sync_copy(gb_hbm, gb_v.at[pl.ds(0, 1)])
        for cp in idx_cps:
            cp.wait()

        cps = []
        for c in range(n_chunks):
            sl = pl.ds(c * CHUNK, CHUNK)
            cps.append(pltpu.async_copy(ub_hbm.at[u_v.at[c]], ub_v.at[sl], sem))
            cps.append(pltpu.async_copy(ib_hbm.at[i_v.at[c]], ib_v.at[sl], sem))
        for d in range(N_DIM):
            for c in range(n_chunks):
                dsl = pl.ds((d * n_chunks + c) * CHUNK, CHUNK)
                cps.append(pltpu.async_copy(
                    uv_hbm.at[pl.ds(d * n_rows, n_rows)].at[u_v.at[c]],
                    pu_v.at[dsl], sem))
                cps.append(pltpu.async_copy(
                    iv_hbm.at[pl.ds(d * n_rows, n_rows)].at[i_v.at[c]],
                    pi_v.at[dsl], sem))
        for cp in cps:
            cp.wait()

        gb = gb_v[...][0]

        def group(g, carry):
            c = g // gpc
            off = (g % gpc) * L
            bu = ub_v[pl.ds(g * L, L)]
            bi = ib_v[pl.ds(g * L, L)]
            s_u = jnp.zeros((L,), jnp.float32)
            s_i = jnp.zeros((L,), jnp.float32)
            t = jnp.zeros((L,), jnp.float32)
            for d in range(N_DIM):
                lu = pu_v[pl.ds((d * n_chunks + c) * CHUNK + off, L)]
                li = pi_v[pl.ds((d * n_chunks + c) * CHUNK + off, L)]
                eu = jnp.exp(0.5 * lu)
                ei = jnp.exp(0.5 * li)
                s_u = s_u + eu * eu
                s_i = s_i + ei * ei
                t = t + eu * ei
            bc = t * _rsqrt(s_u * s_i)
            z = jnp.maximum(1.0 - bc, 1e-36)
            intx = z * _rsqrt(z)
            out_v[pl.ds(g * L, L)] = bu + bi + intx + gb
            return carry

        lax.fori_loop(0, n_groups, group, 0)
        pltpu.sync_copy(out_v, out_hbm.at[pl.ds(base, bpw)])

    return k


def kernel(u, i, user_bias, user_vect, item_bias, item_vect, glob_bias):
    batch = u.shape[0]
    n_rows = user_vect.shape[0]
    k = _make_kernel(batch, n_rows)
    return k(u.astype(jnp.int32), i.astype(jnp.int32),
             user_bias.reshape(-1), user_vect.T.reshape(-1),
             item_bias.reshape(-1), item_vect.T.reshape(-1), glob_bias)


# speedup vs baseline: 12.7270x; 12.7270x over previous
import functools

import jax
import jax.numpy as jnp
from jax import lax
from jax.experimental import pallas as pl
from jax.experimental.pallas import tpu as pltpu
from jax.experimental.pallas import tpu_sc as plsc

N_DIM = 16
L = 16
CHUNK = 128
SUBL = 8
LANE = 128


def _rsqrt(x):
    xi = plsc.bitcast(x, jnp.int32)
    y = plsc.bitcast(jnp.int32(0x5F3759DF) - (xi >> 1), jnp.float32)
    for _ in range(3):
        y = y * (1.5 - 0.5 * x * y * y)
    return y


def _tiled_flat(table):
    v = table.shape[0]
    vp = (v + LANE - 1) // LANE * LANE
    n_tiles = vp // LANE
    padded = jnp.pad(table, ((0, vp - v), (0, 0)))
    x = padded.T.reshape(N_DIM // SUBL, SUBL, n_tiles, LANE)
    return x.transpose(0, 2, 1, 3).reshape(-1), n_tiles


def _make_kernel(batch, n_tiles):
    info = plsc.get_sparse_core_info()
    nc, ns = info.num_cores, info.num_subcores
    nw = nc * ns
    assert batch % (nw * L) == 0
    bpw = batch // nw
    n_chunks = bpw // CHUNK
    n_groups = bpw // L
    gpc = CHUNK // L
    band = n_tiles * SUBL * LANE
    flat_len = (N_DIM // SUBL) * band

    mesh = plsc.VectorSubcoreMesh(core_axis_name="c", subcore_axis_name="s")

    @functools.partial(
        pl.kernel,
        mesh=mesh,
        compiler_params=pltpu.CompilerParams(
            needs_layout_passes=False, use_tc_tiling_on_sc=False),
        out_type=jax.ShapeDtypeStruct((batch,), jnp.float32),
        scratch_types=[
            pltpu.VMEM((n_chunks, CHUNK), jnp.int32),
            pltpu.VMEM((n_chunks, CHUNK), jnp.int32),
            pltpu.VMEM((n_chunks, CHUNK), jnp.int32),
            pltpu.VMEM((n_chunks, CHUNK), jnp.int32),
            pltpu.VMEM((N_DIM * bpw,), jnp.float32),
            pltpu.VMEM((N_DIM * bpw,), jnp.float32),
            pltpu.VMEM((bpw,), jnp.float32),
            pltpu.VMEM((bpw,), jnp.float32),
            pltpu.VMEM((L,), jnp.float32),
            pltpu.VMEM((bpw,), jnp.float32),
            pltpu.SemaphoreType.DMA,
        ],
    )
    def k(u_hbm, i_hbm, ub_hbm, uv_hbm, ib_hbm, iv_hbm, gb_hbm, out_hbm,
          u_v, i_v, bu_v, bi_v, pu_v, pi_v, ub_v, ib_v, gb_v, out_v, sem):
        wid = lax.axis_index("s") * nc + lax.axis_index("c")
        base = wid * bpw

        idx_cps = []
        for c in range(n_chunks):
            idx_cps.append(pltpu.async_copy(
                u_hbm.at[pl.ds(base + c * CHUNK, CHUNK)], u_v.at[c], sem))
            idx_cps.append(pltpu.async_copy(
                i_hbm.at[pl.ds(base + c * CHUNK, CHUNK)], i_v.at[c], sem))
        pltpu.sync_copy(gb_hbm, gb_v.at[pl.ds(0, 1)])
        for cp in idx_cps:
            cp.wait()

        for c in range(n_chunks):
            for j in range(gpc):
                sl = pl.ds(j * L, L)
                uvec = u_v[c, sl]
                ivec = i_v[c, sl]
                bu_v[c, sl] = (uvec >> 7) * (SUBL * LANE) + (uvec & (LANE - 1))
                bi_v[c, sl] = (ivec >> 7) * (SUBL * LANE) + (ivec & (LANE - 1))

        cps = []
        for c in range(n_chunks):
            sl = pl.ds(c * CHUNK, CHUNK)
            cps.append(pltpu.async_copy(ub_hbm.at[u_v.at[c]], ub_v.at[sl], sem))
            cps.append(pltpu.async_copy(ib_hbm.at[i_v.at[c]], ib_v.at[sl], sem))
        for d in range(N_DIM):
            off = (d // SUBL) * band + (d % SUBL) * LANE
            span = flat_len - off
            for c in range(n_chunks):
                dsl = pl.ds((d * n_chunks + c) * CHUNK, CHUNK)
                cps.append(pltpu.async_copy(
                    uv_hbm.at[pl.ds(off, span)].at[bu_v.at[c]],
                    pu_v.at[dsl], sem))
                cps.append(pltpu.async_copy(
                    iv_hbm.at[pl.ds(off, span)].at[bi_v.at[c]],
                    pi_v.at[dsl], sem))
        for cp in cps:
            cp.wait()

        gb = gb_v[...][0]

        def group(g, carry):
            c = g // gpc
            off = (g % gpc) * L
            bu = ub_v[pl.ds(g * L, L)]
            bi = ib_v[pl.ds(g * L, L)]
            s_u = jnp.zeros((L,), jnp.float32)
            s_i = jnp.zeros((L,), jnp.float32)
            t = jnp.zeros((L,), jnp.float32)
            for d in range(N_DIM):
                lu = pu_v[pl.ds((d * n_chunks + c) * CHUNK + off, L)]
                li = pi_v[pl.ds((d * n_chunks + c) * CHUNK + off, L)]
                eu = jnp.exp(0.5 * lu)
                ei = jnp.exp(0.5 * li)
                s_u = s_u + eu * eu
                s_i = s_i + ei * ei
                t = t + eu * ei
            bc = t * _rsqrt(s_u * s_i)
            z = jnp.maximum(1.0 - bc, 1e-36)
            intx = z * _rsqrt(z)
            out_v[pl.ds(g * L, L)] = bu + bi + intx + gb
            return carry

        lax.fori_loop(0, n_groups, group, 0)
        pltpu.sync_copy(out_v, out_hbm.at[pl.ds(base, bpw)])

    return k


def kernel(u, i, user_bias, user_vect, item_bias, item_vect, glob_bias):
    batch = u.shape[0]
    uv_flat, n_tiles = _tiled_flat(user_vect)
    iv_flat, _ = _tiled_flat(item_vect)
    k = _make_kernel(batch, n_tiles)
    return k(u.astype(jnp.int32), i.astype(jnp.int32),
             user_bias.reshape(-1), uv_flat,
             item_bias.reshape(-1), iv_flat, glob_bias)


# speedup vs baseline: 19.1401x; 1.5039x over previous
import functools

import jax
import jax.numpy as jnp
from jax import lax
from jax.experimental import pallas as pl
from jax.experimental.pallas import tpu as pltpu
from jax.experimental.pallas import tpu_sc as plsc

N_DIM = 16
L = 16
CHUNK = 128
SUBL = 8
LANE = 128


def _rsqrt(x):
    xi = plsc.bitcast(x, jnp.int32)
    y = plsc.bitcast(jnp.int32(0x5F3759DF) - (xi >> 1), jnp.float32)
    for _ in range(3):
        y = y * (1.5 - 0.5 * x * y * y)
    return y


def _tiled_flat(table):
    v = table.shape[0]
    vp = (v + LANE - 1) // LANE * LANE
    n_tiles = vp // LANE
    padded = jnp.pad(table, ((0, vp - v), (0, 0)))
    x = padded.T.reshape(N_DIM // SUBL, SUBL, n_tiles, LANE)
    return x.transpose(0, 2, 1, 3).reshape(-1), n_tiles


def _make_kernel(batch, n_tiles):
    info = plsc.get_sparse_core_info()
    nc, ns = info.num_cores, info.num_subcores
    nw = nc * ns
    assert batch % (nw * L) == 0
    bpw = batch // nw
    n_chunks = bpw // CHUNK
    n_groups = bpw // L
    gpc = CHUNK // L
    band = n_tiles * SUBL * LANE
    flat_len = (N_DIM // SUBL) * band

    mesh = plsc.VectorSubcoreMesh(core_axis_name="c", subcore_axis_name="s")

    @functools.partial(
        pl.kernel,
        mesh=mesh,
        compiler_params=pltpu.CompilerParams(
            needs_layout_passes=False, use_tc_tiling_on_sc=False),
        out_type=jax.ShapeDtypeStruct((batch,), jnp.float32),
        scratch_types=[
            pltpu.VMEM((n_chunks, CHUNK), jnp.int32),
            pltpu.VMEM((n_chunks, CHUNK), jnp.int32),
            pltpu.VMEM((n_chunks, CHUNK), jnp.int32),
            pltpu.VMEM((n_chunks, CHUNK), jnp.int32),
            pltpu.VMEM((N_DIM * bpw,), jnp.float32),
            pltpu.VMEM((N_DIM * bpw,), jnp.float32),
            pltpu.VMEM((bpw,), jnp.float32),
            pltpu.VMEM((bpw,), jnp.float32),
            pltpu.VMEM((L,), jnp.float32),
            pltpu.VMEM((bpw,), jnp.float32),
            pltpu.SemaphoreType.DMA,
        ],
    )
    def k(u_hbm, i_hbm, ub_hbm, uv_hbm, ib_hbm, iv_hbm, gb_hbm, out_hbm,
          u_v, i_v, bu_v, bi_v, pu_v, pi_v, ub_v, ib_v, gb_v, out_v, sem):
        wid = lax.axis_index("s") * nc + lax.axis_index("c")
        base = wid * bpw

        idx_cps = []
        for c in range(n_chunks):
            idx_cps.append(pltpu.async_copy(
                u_hbm.at[pl.ds(base + c * CHUNK, CHUNK)], u_v.at[c], sem))
            idx_cps.append(pltpu.async_copy(
                i_hbm.at[pl.ds(base + c * CHUNK, CHUNK)], i_v.at[c], sem))
        pltpu.sync_copy(gb_hbm, gb_v.at[pl.ds(0, 1)])
        for cp in idx_cps:
            cp.wait()

        for c in range(n_chunks):
            for j in range(gpc):
                sl = pl.ds(j * L, L)
                uvec = u_v[c, sl]
                ivec = i_v[c, sl]
                bu_v[c, sl] = (uvec >> 7) * (SUBL * LANE) + (uvec & (LANE - 1))
                bi_v[c, sl] = (ivec >> 7) * (SUBL * LANE) + (ivec & (LANE - 1))

        cps = []
        for c in range(n_chunks):
            sl = pl.ds(c * CHUNK, CHUNK)
            cps.append(pltpu.async_copy(ub_hbm.at[u_v.at[c]], ub_v.at[sl], sem))
            cps.append(pltpu.async_copy(ib_hbm.at[i_v.at[c]], ib_v.at[sl], sem))
        for d in range(N_DIM):
            off = (d // SUBL) * band + (d % SUBL) * LANE
            span = flat_len - off
            for c in range(n_chunks):
                dsl = pl.ds((d * n_chunks + c) * CHUNK, CHUNK)
                cps.append(pltpu.async_copy(
                    uv_hbm.at[pl.ds(off, span)].at[bu_v.at[c]],
                    pu_v.at[dsl], sem))
                cps.append(pltpu.async_copy(
                    iv_hbm.at[pl.ds(off, span)].at[bi_v.at[c]],
                    pi_v.at[dsl], sem))
        for cp in cps:
            cp.wait()

        gb = gb_v[...][0]

        def group(g, carry):
            c = g // gpc
            off = (g % gpc) * L
            bu = ub_v[pl.ds(g * L, L)]
            bi = ib_v[pl.ds(g * L, L)]
            s_u = jnp.zeros((L,), jnp.float32)
            s_i = jnp.zeros((L,), jnp.float32)
            t = jnp.zeros((L,), jnp.float32)
            for d in range(N_DIM):
                lu = pu_v[pl.ds((d * n_chunks + c) * CHUNK + off, L)]
                li = pi_v[pl.ds((d * n_chunks + c) * CHUNK + off, L)]
                eu = jnp.exp(0.5 * lu)
                ei = jnp.exp(0.5 * li)
                s_u = s_u + eu * eu
                s_i = s_i + ei * ei
                t = t + eu * ei
            bc = t * _rsqrt(s_u * s_i)
            z = jnp.maximum(1.0 - bc, 1e-36)
            intx = z * _rsqrt(z)
            out_v[pl.ds(g * L, L)] = bu + bi + intx + gb
            return carry

        lax.fori_loop(0, n_groups, group, 0)
        pltpu.sync_copy(out_v, out_hbm.at[pl.ds(base, bpw)])

    return k


def _bias_flat(bias):
    v = bias.shape[0]
    vp = (v + 1023) // 1024 * 1024
    return jnp.pad(bias, ((0, vp - v), (0, 0))).reshape(-1)


def kernel(u, i, user_bias, user_vect, item_bias, item_vect, glob_bias):
    batch = u.shape[0]
    uv_flat, n_tiles = _tiled_flat(user_vect)
    iv_flat, _ = _tiled_flat(item_vect)
    k = _make_kernel(batch, n_tiles)
    return k(u.astype(jnp.int32), i.astype(jnp.int32),
             _bias_flat(user_bias), uv_flat,
             _bias_flat(item_bias), iv_flat, glob_bias)


# speedup vs baseline: 19.2662x; 1.0066x over previous
import functools

import jax
import jax.numpy as jnp
from jax import lax
from jax.experimental import pallas as pl
from jax.experimental.pallas import tpu as pltpu
from jax.experimental.pallas import tpu_sc as plsc

N_DIM = 16
L = 16
SUBL = 8
LANE = 128


def _rsqrt(x):
    xi = plsc.bitcast(x, jnp.int32)
    y = plsc.bitcast(jnp.int32(0x5F3759DF) - (xi >> 1), jnp.float32)
    for _ in range(3):
        y = y * (1.5 - 0.5 * x * y * y)
    return y


def _tiled_flat(table):
    v = table.shape[0]
    vp = (v + LANE - 1) // LANE * LANE
    n_tiles = vp // LANE
    padded = jnp.pad(table, ((0, vp - v), (0, 0)))
    x = padded.T.reshape(N_DIM // SUBL, SUBL, n_tiles, LANE)
    return x.transpose(0, 2, 1, 3).reshape(-1), n_tiles


def _bias_flat(bias):
    v = bias.shape[0]
    vp = (v + 1023) // 1024 * 1024
    return jnp.pad(bias, ((0, vp - v), (0, 0))).reshape(-1)


def _make_kernel(batch, n_tiles):
    info = plsc.get_sparse_core_info()
    nc, ns = info.num_cores, info.num_subcores
    nw = nc * ns
    assert batch % (nw * L) == 0
    bpw = batch // nw
    n_groups = bpw // L
    band = n_tiles * SUBL * LANE
    flat_len = (N_DIM // SUBL) * band

    mesh = plsc.VectorSubcoreMesh(core_axis_name="c", subcore_axis_name="s")

    @functools.partial(
        pl.kernel,
        mesh=mesh,
        compiler_params=pltpu.CompilerParams(
            needs_layout_passes=False, use_tc_tiling_on_sc=False),
        out_type=jax.ShapeDtypeStruct((batch,), jnp.float32),
        scratch_types=[
            pltpu.VMEM((bpw,), jnp.int32),
            pltpu.VMEM((bpw,), jnp.int32),
            pltpu.VMEM((bpw,), jnp.int32),
            pltpu.VMEM((bpw,), jnp.int32),
            pltpu.VMEM((N_DIM * bpw,), jnp.float32),
            pltpu.VMEM((N_DIM * bpw,), jnp.float32),
            pltpu.VMEM((bpw,), jnp.float32),
            pltpu.VMEM((bpw,), jnp.float32),
            pltpu.VMEM((L,), jnp.float32),
            pltpu.VMEM((bpw,), jnp.float32),
            pltpu.SemaphoreType.DMA,
        ],
    )
    def k(u_hbm, i_hbm, ub_hbm, uv_hbm, ib_hbm, iv_hbm, gb_hbm, out_hbm,
          u_v, i_v, tu_v, ti_v, pu_v, pi_v, ub_v, ib_v, gb_v, out_v, sem):
        wid = lax.axis_index("s") * nc + lax.axis_index("c")
        base = wid * bpw

        cp_u = pltpu.async_copy(u_hbm.at[pl.ds(base, bpw)], u_v, sem)
        cp_i = pltpu.async_copy(i_hbm.at[pl.ds(base, bpw)], i_v, sem)
        pltpu.sync_copy(gb_hbm, gb_v.at[pl.ds(0, 1)])
        cp_u.wait()
        cp_i.wait()

        for j in range(bpw // L):
            sl = pl.ds(j * L, L)
            uvec = u_v[sl]
            ivec = i_v[sl]
            tu_v[sl] = (uvec >> 7) * (SUBL * LANE) + (uvec & (LANE - 1))
            ti_v[sl] = (ivec >> 7) * (SUBL * LANE) + (ivec & (LANE - 1))

        cps = [
            pltpu.async_copy(ub_hbm.at[u_v], ub_v, sem),
            pltpu.async_copy(ib_hbm.at[i_v], ib_v, sem),
        ]
        for d in range(N_DIM):
            off = (d // SUBL) * band + (d % SUBL) * LANE
            span = flat_len - off
            cps.append(pltpu.async_copy(
                uv_hbm.at[pl.ds(off, span)].at[tu_v],
                pu_v.at[pl.ds(d * bpw, bpw)], sem))
            cps.append(pltpu.async_copy(
                iv_hbm.at[pl.ds(off, span)].at[ti_v],
                pi_v.at[pl.ds(d * bpw, bpw)], sem))
        for cp in cps:
            cp.wait()

        gb = gb_v[...][0]

        def group(g, carry):
            bu = ub_v[pl.ds(g * L, L)]
            bi = ib_v[pl.ds(g * L, L)]
            s_u = jnp.zeros((L,), jnp.float32)
            s_i = jnp.zeros((L,), jnp.float32)
            t = jnp.zeros((L,), jnp.float32)
            for d in range(N_DIM):
                lu = pu_v[pl.ds(d * bpw + g * L, L)]
                li = pi_v[pl.ds(d * bpw + g * L, L)]
                eu = jnp.exp(0.5 * lu)
                ei = jnp.exp(0.5 * li)
                s_u = s_u + eu * eu
                s_i = s_i + ei * ei
                t = t + eu * ei
            bc = t * _rsqrt(s_u * s_i)
            z = jnp.maximum(1.0 - bc, 1e-36)
            intx = z * _rsqrt(z)
            out_v[pl.ds(g * L, L)] = bu + bi + intx + gb
            return carry

        lax.fori_loop(0, n_groups, group, 0)
        pltpu.sync_copy(out_v, out_hbm.at[pl.ds(base, bpw)])

    return k


def kernel(u, i, user_bias, user_vect, item_bias, item_vect, glob_bias):
    batch = u.shape[0]
    uv_flat, n_tiles = _tiled_flat(user_vect)
    iv_flat, _ = _tiled_flat(item_vect)
    k = _make_kernel(batch, n_tiles)
    return k(u.astype(jnp.int32), i.astype(jnp.int32),
             _bias_flat(user_bias), uv_flat,
             _bias_flat(item_bias), iv_flat, glob_bias)
